# baseline (speedup 1.0000x reference)
import functools

import jax
import jax.numpy as jnp
from jax import lax
from jax.experimental import pallas as pl
from jax.experimental.pallas import tpu as pltpu
from jax.experimental.pallas import tpu_sc as plsc

NC = 2
NS = 16
NW = NC * NS
WIN = 64
NBUF = 4
CHW = 16
ROW_BLK = 2000


def _msg_body(x_ref, wg_ref, we_ref, o_ref):
    x = x_ref[...]
    g = jnp.dot(x, wg_ref[...], preferred_element_type=jnp.float32)
    e = jnp.dot(x, we_ref[...], preferred_element_type=jnp.float32)
    o_ref[...] = jax.nn.sigmoid(g) * e


def _messages(x, wg_t, we_t):
    n, d = x.shape
    return pl.pallas_call(
        _msg_body,
        grid=(n // ROW_BLK,),
        in_specs=[
            pl.BlockSpec((ROW_BLK, d), lambda i: (i, 0)),
            pl.BlockSpec((d, d), lambda i: (0, 0)),
            pl.BlockSpec((d, d), lambda i: (0, 0)),
        ],
        out_specs=pl.BlockSpec((ROW_BLK, d), lambda i: (i, 0)),
        out_shape=jax.ShapeDtypeStruct((n, d), jnp.float32),
    )(x, wg_t, we_t)


def _combine_body(x_ref, p_ref, o_ref):
    o_ref[...] = x_ref[...] + p_ref[0] + p_ref[1]


def _combine(x, p):
    n, d = x.shape
    return pl.pallas_call(
        _combine_body,
        grid=(n // ROW_BLK,),
        in_specs=[
            pl.BlockSpec((ROW_BLK, d), lambda i: (i, 0)),
            pl.BlockSpec((2, ROW_BLK, d), lambda i: (0, i, 0)),
        ],
        out_specs=pl.BlockSpec((ROW_BLK, d), lambda i: (i, 0)),
        out_shape=jax.ShapeDtypeStruct((n, d), jnp.float32),
    )(x, p)


def _sc_partials(msg, src3, tgt3, n_nodes, acc_rows, a_win, b_win):
    kwin = src3.shape[1]
    d = msg.shape[1]
    rows_per = acc_rows // NS
    zch = acc_rows // NS // WIN
    mesh = plsc.VectorSubcoreMesh(core_axis_name="c", subcore_axis_name="s")

    @functools.partial(
        pl.kernel,
        mesh=mesh,
        out_type=jax.ShapeDtypeStruct((NC, acc_rows, d), jnp.float32),
        scratch_types=[
            pltpu.VMEM((2, CHW, WIN), jnp.int32),
            pltpu.VMEM((2, CHW, WIN), jnp.int32),
            pltpu.VMEM((NBUF, WIN, d), jnp.float32),
            pltpu.VMEM_SHARED((acc_rows, d), jnp.float32),
            pltpu.SemaphoreType.DMA,
            pltpu.SemaphoreType.DMA,
            pltpu.SemaphoreType.DMA,
            pltpu.SemaphoreType.DMA,
            pltpu.SemaphoreType.DMA,
            pltpu.SemaphoreType.DMA,
            pltpu.SemaphoreType.DMA,
            pltpu.SemaphoreType.DMA,
            pltpu.SemaphoreType.DMA,
        ],
    )
    def k_fn(msg_hbm, src_hbm, tgt_hbm, out_hbm, src_v, tgt_v, bufs, acc,
             sem0, sem1, sem2, sem3, ssem0, ssem1, ssem2, ssem3, isem):
        cid = lax.axis_index("c")
        sid = lax.axis_index("s")
        wid = cid * NS + sid
        sems = (sem0, sem1, sem2, sem3)
        ssems = (ssem0, ssem1, ssem2, ssem3)
        buf0 = bufs.at[0]

        with jax.named_scope("zero_fill"):
            @pl.loop(0, WIN)
            def _(i):
                @pl.loop(0, d, step=16)
                def _(jj):
                    bufs[0, i, pl.ds(jj, 16)] = jnp.zeros((16,), jnp.float32)

            zbase = sid * (acc_rows // NS)

            @pl.loop(0, zch)
            def _(i):
                pltpu.sync_copy(buf0, acc.at[pl.ds(zbase + i * WIN, WIN)])

            plsc.subcore_barrier()

        my_src = src_hbm.at[wid]
        my_tgt = tgt_hbm.at[wid]
        with jax.named_scope("idx_stage0"):
            pltpu.sync_copy(my_src.at[pl.ds(0, CHW)], src_v.at[0])
            pltpu.sync_copy(my_tgt.at[pl.ds(0, CHW)], tgt_v.at[0])

        nch = kwin // CHW

        with jax.named_scope("gather_scatter"):
            for w0 in range(2):
                pltpu.async_copy(
                    msg_hbm.at[tgt_v.at[0].at[w0]], bufs.at[w0], sems[w0]
                )

            @pl.loop(0, kwin, step=NBUF)
            def _(j):
                for b in range(NBUF):
                    w = j + b
                    cw = w // CHW
                    par = cw % 2

                    pltpu.make_async_copy(
                        msg_hbm.at[tgt_v.at[0].at[0]], bufs.at[b], sems[b]
                    ).wait()

                    bp = (b + 3) % NBUF

                    @pl.when(w >= 1)
                    def _():
                        pltpu.make_async_copy(
                            bufs.at[bp], acc.at[src_v.at[0].at[0]], ssems[bp]
                        ).wait()

                    pltpu.async_copy(
                        bufs.at[b], acc.at[src_v.at[par].at[w % CHW]],
                        ssems[b], add=True
                    )

                    @pl.when((w % CHW == 2) & (cw + 1 < nch))
                    def _():
                        pltpu.async_copy(
                            my_src.at[pl.ds((cw + 1) * CHW, CHW)],
                            src_v.at[1 - par], isem
                        )
                        pltpu.async_copy(
                            my_tgt.at[pl.ds((cw + 1) * CHW, CHW)],
                            tgt_v.at[1 - par], isem
                        )

                    @pl.when((w % CHW == CHW - 2) & (cw + 1 < nch))
                    def _():
                        pltpu.make_async_copy(
                            my_src.at[pl.ds(0, CHW)], src_v.at[1 - par], isem
                        ).wait()
                        pltpu.make_async_copy(
                            my_tgt.at[pl.ds(0, CHW)], tgt_v.at[1 - par], isem
                        ).wait()

                    bn = (b + 2) % NBUF

                    @pl.when(w + 2 < kwin)
                    def _():
                        t = w + 2
                        pt = (t // CHW) % 2
                        pltpu.async_copy(
                            msg_hbm.at[tgt_v.at[pt].at[t % CHW]],
                            bufs.at[bn], sems[bn]
                        )

            pltpu.make_async_copy(
                bufs.at[3], acc.at[src_v.at[0].at[0]], ssems[3]
            ).wait()

        plsc.subcore_barrier()

        with jax.named_scope("copy_out"):
            rbase = sid * rows_per
            pltpu.sync_copy(
                acc.at[pl.ds(rbase, rows_per)],
                out_hbm.at[cid].at[pl.ds(rbase, rows_per)],
            )

    return k_fn(msg, src3, tgt3)


def kernel(input, edge_sources, edge_targets, W):
    x = input
    n, d = x.shape
    dout = W.shape[0] // 2
    wg_t = W[:dout].T
    we_t = W[dout:].T
    msg = _messages(x, wg_t, we_t)

    e = edge_sources.shape[0]
    kwin = -(-e // (NW * WIN))
    kwin = -(-kwin // CHW) * CHW
    epad = NW * kwin * WIN
    acc_rows = -(-(n + WIN) // (NS * WIN)) * (NS * WIN)
    lanes = jax.lax.iota(jnp.int32, epad - e) % WIN
    src = jnp.concatenate([edge_sources.astype(jnp.int32), n + lanes])
    tgt = jnp.concatenate([edge_targets.astype(jnp.int32), lanes])
    partial = _sc_partials(
        msg,
        src.reshape(NW, kwin, WIN),
        tgt.reshape(NW, kwin, WIN),
        n,
        acc_rows,
        kwin,
        kwin,
    )
    return _combine(x, partial)


# speedup vs baseline: 1.1208x; 1.1208x over previous
import functools

import jax
import jax.numpy as jnp
from jax import lax
from jax.experimental import pallas as pl
from jax.experimental.pallas import tpu as pltpu
from jax.experimental.pallas import tpu_sc as plsc

NC = 2
NS = 16
NW = NC * NS
WIN = 64
NBUF = 4
CHW = 16
ROW_BLK = 2000


def _msg_body(x_ref, wg_ref, we_ref, o_ref):
    x = x_ref[...]
    g = jnp.dot(x, wg_ref[...], preferred_element_type=jnp.float32)
    e = jnp.dot(x, we_ref[...], preferred_element_type=jnp.float32)
    o_ref[...] = jax.nn.sigmoid(g) * e


def _messages(x, wg_t, we_t):
    n, d = x.shape
    return pl.pallas_call(
        _msg_body,
        grid=(n // ROW_BLK,),
        in_specs=[
            pl.BlockSpec((ROW_BLK, d), lambda i: (i, 0)),
            pl.BlockSpec((d, d), lambda i: (0, 0)),
            pl.BlockSpec((d, d), lambda i: (0, 0)),
        ],
        out_specs=pl.BlockSpec((ROW_BLK, d), lambda i: (i, 0)),
        out_shape=jax.ShapeDtypeStruct((n, d), jnp.float32),
    )(x, wg_t, we_t)


def _combine_body(x_ref, p_ref, o_ref):
    o_ref[...] = x_ref[...] + p_ref[0] + p_ref[1]


def _combine(x, p):
    n, d = x.shape
    return pl.pallas_call(
        _combine_body,
        grid=(n // ROW_BLK,),
        in_specs=[
            pl.BlockSpec((ROW_BLK, d), lambda i: (i, 0)),
            pl.BlockSpec((2, ROW_BLK, d), lambda i: (0, i, 0)),
        ],
        out_specs=pl.BlockSpec((ROW_BLK, d), lambda i: (i, 0)),
        out_shape=jax.ShapeDtypeStruct((n, d), jnp.float32),
    )(x, p)


def _sc_partials(msg, src3, tgt3, n_nodes, acc_rows, a_win, b_win):
    kwin = src3.shape[1]
    d = msg.shape[1]
    rows_per = acc_rows // NS
    zch = acc_rows // NS // WIN
    mesh = plsc.VectorSubcoreMesh(core_axis_name="c", subcore_axis_name="s")

    @functools.partial(
        pl.kernel,
        mesh=mesh,
        out_type=jax.ShapeDtypeStruct((NC, acc_rows, d), jnp.float32),
        scratch_types=[
            pltpu.VMEM((2, CHW, WIN), jnp.int32),
            pltpu.VMEM((2, CHW, WIN), jnp.int32),
            pltpu.VMEM((NBUF, WIN, d), jnp.float32),
            pltpu.VMEM_SHARED((acc_rows, d), jnp.float32),
            pltpu.SemaphoreType.DMA,
            pltpu.SemaphoreType.DMA,
            pltpu.SemaphoreType.DMA,
            pltpu.SemaphoreType.DMA,
            pltpu.SemaphoreType.DMA,
            pltpu.SemaphoreType.DMA,
            pltpu.SemaphoreType.DMA,
            pltpu.SemaphoreType.DMA,
            pltpu.SemaphoreType.DMA,
        ],
    )
    def k_fn(msg_hbm, src_hbm, tgt_hbm, out_hbm, src_v, tgt_v, bufs, acc,
             sem0, sem1, sem2, sem3, ssem0, ssem1, ssem2, ssem3, isem):
        cid = lax.axis_index("c")
        sid = lax.axis_index("s")
        wid = cid * NS + sid
        sems = (sem0, sem1, sem2, sem3)
        ssems = (ssem0, ssem1, ssem2, ssem3)
        buf0 = bufs.at[0]

        with jax.named_scope("zero_fill"):
            @pl.loop(0, WIN)
            def _(i):
                @pl.loop(0, d, step=16)
                def _(jj):
                    bufs[0, i, pl.ds(jj, 16)] = jnp.zeros((16,), jnp.float32)

            zbase = sid * (acc_rows // NS)

            @pl.loop(0, zch)
            def _(i):
                pltpu.sync_copy(buf0, acc.at[pl.ds(zbase + i * WIN, WIN)])

            plsc.subcore_barrier()

        my_src = src_hbm.at[wid]
        my_tgt = tgt_hbm.at[wid]
        with jax.named_scope("idx_stage0"):
            pltpu.sync_copy(my_src.at[pl.ds(0, CHW)], src_v.at[0])
            pltpu.sync_copy(my_tgt.at[pl.ds(0, CHW)], tgt_v.at[0])

        nch = kwin // CHW

        with jax.named_scope("gather_scatter"):
            for w0 in range(3):
                pltpu.async_copy(
                    msg_hbm.at[tgt_v.at[0].at[w0]], bufs.at[w0], sems[w0]
                )

            @pl.loop(0, kwin, step=NBUF)
            def _(j):
                for b in range(NBUF):
                    w = j + b
                    cw = w // CHW
                    par = cw % 2

                    pltpu.make_async_copy(
                        msg_hbm.at[tgt_v.at[0].at[0]], bufs.at[b], sems[b]
                    ).wait()

                    bp = (b + 3) % NBUF

                    @pl.when(w >= 1)
                    def _():
                        pltpu.make_async_copy(
                            bufs.at[bp], acc.at[src_v.at[0].at[0]], ssems[bp]
                        ).wait()

                    pltpu.async_copy(
                        bufs.at[b], acc.at[src_v.at[par].at[w % CHW]],
                        ssems[b], add=True
                    )

                    @pl.when((w % CHW == 2) & (cw + 1 < nch))
                    def _():
                        pltpu.async_copy(
                            my_src.at[pl.ds((cw + 1) * CHW, CHW)],
                            src_v.at[1 - par], isem
                        )
                        pltpu.async_copy(
                            my_tgt.at[pl.ds((cw + 1) * CHW, CHW)],
                            tgt_v.at[1 - par], isem
                        )

                    @pl.when((w % CHW == CHW - 4) & (cw + 1 < nch))
                    def _():
                        pltpu.make_async_copy(
                            my_src.at[pl.ds(0, CHW)], src_v.at[1 - par], isem
                        ).wait()
                        pltpu.make_async_copy(
                            my_tgt.at[pl.ds(0, CHW)], tgt_v.at[1 - par], isem
                        ).wait()

                    bn = (b + 3) % NBUF

                    @pl.when(w + 3 < kwin)
                    def _():
                        t = w + 3
                        pt = (t // CHW) % 2
                        pltpu.async_copy(
                            msg_hbm.at[tgt_v.at[pt].at[t % CHW]],
                            bufs.at[bn], sems[bn]
                        )

            pltpu.make_async_copy(
                bufs.at[3], acc.at[src_v.at[0].at[0]], ssems[3]
            ).wait()

        plsc.subcore_barrier()

        with jax.named_scope("copy_out"):
            rbase = sid * rows_per
            pltpu.sync_copy(
                acc.at[pl.ds(rbase, rows_per)],
                out_hbm.at[cid].at[pl.ds(rbase, rows_per)],
            )

    return k_fn(msg, src3, tgt3)


def kernel(input, edge_sources, edge_targets, W):
    x = input
    n, d = x.shape
    dout = W.shape[0] // 2
    wg_t = W[:dout].T
    we_t = W[dout:].T
    msg = _messages(x, wg_t, we_t)

    e = edge_sources.shape[0]
    kwin = -(-e // (NW * WIN))
    kwin = -(-kwin // CHW) * CHW
    epad = NW * kwin * WIN
    acc_rows = -(-(n + WIN) // (NS * WIN)) * (NS * WIN)
    lanes = jax.lax.iota(jnp.int32, epad - e) % WIN
    src = jnp.concatenate([edge_sources.astype(jnp.int32), n + lanes])
    tgt = jnp.concatenate([edge_targets.astype(jnp.int32), lanes])
    partial = _sc_partials(
        msg,
        src.reshape(NW, kwin, WIN),
        tgt.reshape(NW, kwin, WIN),
        n,
        acc_rows,
        kwin,
        kwin,
    )
    return _combine(x, partial)
